# baseline (speedup 1.0000x reference)
import jax
import jax.numpy as jnp
from jax.experimental import pallas as pl
from jax.experimental.pallas import tpu as pltpu

_BS, _L, _D = 128, 12, 8192
_NC = 8
_CB = _BS // _NC


def _body(idx_ref, drop_ref, in_hbm, out_hbm, zrow, bulk_sems, row_sem):
    zrow[...] = jnp.zeros((1, _D), jnp.float32)

    def _bulk(c):
        return pltpu.make_async_copy(
            in_hbm.at[pl.ds(c * _CB, _CB)],
            out_hbm.at[pl.ds(c * _CB, _CB)],
            bulk_sems.at[c],
        )

    for c in range(_NC):
        _bulk(c).start()

    def _row_copy(b, idx):
        return pltpu.make_async_copy(
            zrow, out_hbm.at[b, pl.ds(idx, 1)], row_sem
        )

    def _chunk_rows(c, start):
        def body(k, _):
            b = c * _CB + k
            dropped = drop_ref[b] > (1.0 - 0.2)

            @pl.when(dropped)
            def _():
                cp = _row_copy(b, idx_ref[b])
                if start:
                    cp.start()
                else:
                    cp.wait()

            return 0

        jax.lax.fori_loop(0, _CB, body, 0)

    for c in range(_NC):
        _bulk(c).wait()
        _chunk_rows(c, start=True)
    for c in range(_NC):
        _chunk_rows(c, start=False)


@jax.jit
def kernel(inputs, index, drop_rand):
    return pl.pallas_call(
        _body,
        grid=(),
        in_specs=[
            pl.BlockSpec(memory_space=pltpu.SMEM),
            pl.BlockSpec(memory_space=pltpu.SMEM),
            pl.BlockSpec(memory_space=pl.ANY),
        ],
        out_specs=pl.BlockSpec(memory_space=pl.ANY),
        out_shape=jax.ShapeDtypeStruct((_BS, _L, _D), jnp.float32),
        scratch_shapes=[
            pltpu.VMEM((1, _D), jnp.float32),
            pltpu.SemaphoreType.DMA((_NC,)),
            pltpu.SemaphoreType.DMA,
        ],
    )(index, drop_rand, inputs)


# speedup vs baseline: 16.5696x; 16.5696x over previous
import jax
import jax.numpy as jnp
from jax.experimental import pallas as pl
from jax.experimental.pallas import tpu as pltpu

_BS, _L, _D = 128, 12, 8192
_CB = 8
_NC = _BS // _CB
_NB = 6
_LA = 3


def _body(idx_ref, drop_ref, in_hbm, out_hbm, bufs, in_sems, out_sems):
    def _in(c):
        return pltpu.make_async_copy(
            in_hbm.at[pl.ds(c * _CB, _CB)], bufs.at[c % _NB], in_sems.at[c % _NB]
        )

    def _out(c):
        return pltpu.make_async_copy(
            bufs.at[c % _NB], out_hbm.at[pl.ds(c * _CB, _CB)], out_sems.at[c % _NB]
        )

    def _fix(c):
        def body(k, _):
            b = c * _CB + k
            dropped = drop_ref[b] > (1.0 - 0.2)

            @pl.when(dropped)
            def _():
                bufs[c % _NB, k, pl.ds(idx_ref[b], 1), :] = jnp.zeros(
                    (1, _D), jnp.float32
                )

            return 0

        jax.lax.fori_loop(0, _CB, body, 0)

    for i in range(_LA):
        _in(i).start()
    for c in range(_NC):
        _in(c).wait()
        _fix(c)
        _out(c).start()
        n = c + _LA
        if n < _NC:
            if n - _NB >= 0:
                _out(n - _NB).wait()
            _in(n).start()
    for c in range(_NC - _NB, _NC):
        if c >= 0:
            _out(c).wait()


@jax.jit
def kernel(inputs, index, drop_rand):
    return pl.pallas_call(
        _body,
        grid=(),
        in_specs=[
            pl.BlockSpec(memory_space=pltpu.SMEM),
            pl.BlockSpec(memory_space=pltpu.SMEM),
            pl.BlockSpec(memory_space=pl.ANY),
        ],
        out_specs=pl.BlockSpec(memory_space=pl.ANY),
        out_shape=jax.ShapeDtypeStruct((_BS, _L, _D), jnp.float32),
        scratch_shapes=[
            pltpu.VMEM((_NB, _CB, _L, _D), jnp.float32),
            pltpu.SemaphoreType.DMA((_NB,)),
            pltpu.SemaphoreType.DMA((_NB,)),
        ],
    )(index, drop_rand, inputs)


# speedup vs baseline: 16.5796x; 1.0006x over previous
import jax
import jax.numpy as jnp
from jax.experimental import pallas as pl
from jax.experimental.pallas import tpu as pltpu

_BS, _L, _D = 128, 12, 8192
_CB = 4
_NC = _BS // _CB
_NB = 12
_LA = 6


def _body(idx_ref, drop_ref, in_hbm, out_hbm, bufs, in_sems, out_sems):
    def _in(c):
        return pltpu.make_async_copy(
            in_hbm.at[pl.ds(c * _CB, _CB)], bufs.at[c % _NB], in_sems.at[c % _NB]
        )

    def _out(c):
        return pltpu.make_async_copy(
            bufs.at[c % _NB], out_hbm.at[pl.ds(c * _CB, _CB)], out_sems.at[c % _NB]
        )

    def _fix(c):
        def body(k, _):
            b = c * _CB + k
            dropped = drop_ref[b] > (1.0 - 0.2)

            @pl.when(dropped)
            def _():
                bufs[c % _NB, k, pl.ds(idx_ref[b], 1), :] = jnp.zeros(
                    (1, _D), jnp.float32
                )

            return 0

        jax.lax.fori_loop(0, _CB, body, 0)

    for i in range(_LA):
        _in(i).start()
    for c in range(_NC):
        _in(c).wait()
        _fix(c)
        _out(c).start()
        n = c + _LA
        if n < _NC:
            if n - _NB >= 0:
                _out(n - _NB).wait()
            _in(n).start()
    for c in range(_NC - _NB, _NC):
        if c >= 0:
            _out(c).wait()


@jax.jit
def kernel(inputs, index, drop_rand):
    return pl.pallas_call(
        _body,
        grid=(),
        in_specs=[
            pl.BlockSpec(memory_space=pltpu.SMEM),
            pl.BlockSpec(memory_space=pltpu.SMEM),
            pl.BlockSpec(memory_space=pl.ANY),
        ],
        out_specs=pl.BlockSpec(memory_space=pl.ANY),
        out_shape=jax.ShapeDtypeStruct((_BS, _L, _D), jnp.float32),
        scratch_shapes=[
            pltpu.VMEM((_NB, _CB, _L, _D), jnp.float32),
            pltpu.SemaphoreType.DMA((_NB,)),
            pltpu.SemaphoreType.DMA((_NB,)),
        ],
    )(index, drop_rand, inputs)
